# Initial kernel scaffold; baseline (speedup 1.0000x reference)
import functools

import jax
import jax.numpy as jnp
from jax import lax
from jax.experimental import pallas as pl
from jax.experimental.pallas import tpu as pltpu
from jax.experimental.pallas import tpu_sc as plsc

VOCAB = 1_000_000
D = 64
B = 16384
L = 200
NW = 32
BPW = B // NW
CB = 8
NCHUNK = BPW // CB
IW = 100
GPB = L // IW


def _sc_body(x_hbm, tab_hbm, out_hbm, idx_v, rows_v, out_v, sem):
    cid = lax.axis_index("c")
    sid = lax.axis_index("s")
    wid = sid * 2 + cid
    inv = jnp.float32(1.0 / L)

    def chunk(c, carry):
        bag0 = wid * BPW + c * CB
        r0 = bag0 * GPB
        pltpu.sync_copy(x_hbm.at[pl.ds(r0, CB * GPB)], idx_v)
        cps = []
        for j in range(CB * GPB):
            cps.append(
                pltpu.async_copy(
                    tab_hbm.at[idx_v.at[j]],
                    rows_v.at[pl.ds(j * IW, IW)],
                    sem,
                )
            )
        for cp in cps:
            cp.wait()
        for i in range(CB):
            base = i * L

            def body(r, accs):
                a0, a1, a2, a3 = accs
                row = base + r
                a0 = a0 + rows_v[row, pl.ds(0, 16)]
                a1 = a1 + rows_v[row, pl.ds(16, 16)]
                a2 = a2 + rows_v[row, pl.ds(32, 16)]
                a3 = a3 + rows_v[row, pl.ds(48, 16)]
                return (a0, a1, a2, a3)

            z = jnp.zeros((16,), jnp.float32)
            a0, a1, a2, a3 = lax.fori_loop(0, L, body, (z, z, z, z))
            out_v[i, pl.ds(0, 16)] = a0 * inv
            out_v[i, pl.ds(16, 16)] = a1 * inv
            out_v[i, pl.ds(32, 16)] = a2 * inv
            out_v[i, pl.ds(48, 16)] = a3 * inv
        pltpu.sync_copy(out_v, out_hbm.at[pl.ds(bag0, CB)])
        return carry

    lax.fori_loop(0, NCHUNK, chunk, 0)


def _bag_means(x2d, emb_table):
    mesh = plsc.VectorSubcoreMesh(core_axis_name="c", subcore_axis_name="s")
    k = pl.kernel(
        _sc_body,
        out_type=jax.ShapeDtypeStruct((B, D), jnp.float32),
        mesh=mesh,
        scratch_types=[
            pltpu.VMEM((CB * GPB, IW), jnp.int32),
            pltpu.VMEM((CB * L, D), jnp.float32),
            pltpu.VMEM((CB, D), jnp.float32),
            pltpu.SemaphoreType.DMA,
        ],
    )
    return k(x2d, emb_table)


def _tc_linear(means, Wt, b2):
    def body(s_ref, w_ref, b_ref, o_ref):
        o_ref[...] = (
            jnp.dot(s_ref[...], w_ref[...], preferred_element_type=jnp.float32)
            + b_ref[...]
        )

    BLK = 4096
    return pl.pallas_call(
        body,
        grid=(B // BLK,),
        in_specs=[
            pl.BlockSpec((BLK, D), lambda i: (i, 0)),
            pl.BlockSpec((D, D), lambda i: (0, 0)),
            pl.BlockSpec((1, D), lambda i: (0, 0)),
        ],
        out_specs=pl.BlockSpec((BLK, D), lambda i: (i, 0)),
        out_shape=jax.ShapeDtypeStruct((B, D), jnp.float32),
    )(means, Wt, b2)


def kernel(x, emb_table, W, b):
    x2d = x.astype(jnp.int32).reshape(B * GPB, IW)
    means = _bag_means(x2d, emb_table)
    return _tc_linear(means, W.T, b.reshape(1, D))


# speedup vs baseline: 2.3652x; 2.3652x over previous
import functools

import jax
import jax.numpy as jnp
from jax import lax
from jax.experimental import pallas as pl
from jax.experimental.pallas import tpu as pltpu
from jax.experimental.pallas import tpu_sc as plsc

VOCAB = 1_000_000
D = 64
B = 16384
L = 200
NW = 32
BPW = B // NW
CB = 8
NCHUNK = BPW // CB
IW = 100
GPB = L // IW


def _sc_body(x_hbm, tab_hbm, out_hbm, idx_v, rows_v, out_v, sem):
    cid = lax.axis_index("c")
    sid = lax.axis_index("s")
    wid = sid * 2 + cid
    inv = jnp.float32(1.0 / L)

    def chunk(c, carry):
        bag0 = wid * BPW + c * CB
        r0 = bag0 * GPB
        pltpu.sync_copy(x_hbm.at[pl.ds(r0, CB * GPB)], idx_v)
        cps = []
        for j in range(CB * GPB):
            cps.append(
                pltpu.async_copy(
                    tab_hbm.at[idx_v.at[j]],
                    rows_v.at[pl.ds(j * IW, IW)],
                    sem,
                )
            )
        for cp in cps:
            cp.wait()
        for i in range(CB):
            base = i * L

            def body(r, accs):
                a0, a1, a2, a3 = accs
                row = base + r
                a0 = a0 + rows_v[row, pl.ds(0, 16)]
                a1 = a1 + rows_v[row, pl.ds(16, 16)]
                a2 = a2 + rows_v[row, pl.ds(32, 16)]
                a3 = a3 + rows_v[row, pl.ds(48, 16)]
                return (a0, a1, a2, a3)

            z = jnp.zeros((16,), jnp.float32)
            a0, a1, a2, a3 = lax.fori_loop(0, L, body, (z, z, z, z))
            out_v[i, pl.ds(0, 16)] = a0 * inv
            out_v[i, pl.ds(16, 16)] = a1 * inv
            out_v[i, pl.ds(32, 16)] = a2 * inv
            out_v[i, pl.ds(48, 16)] = a3 * inv
        pltpu.sync_copy(out_v, out_hbm.at[pl.ds(bag0, CB)])
        return carry

    lax.fori_loop(0, NCHUNK, chunk, 0)


def _bag_means(x2d, emb_table):
    mesh = plsc.VectorSubcoreMesh(core_axis_name="c", subcore_axis_name="s")
    k = pl.kernel(
        _sc_body,
        out_type=jax.ShapeDtypeStruct((B, D), jnp.float32),
        mesh=mesh,
        scratch_types=[
            pltpu.VMEM((CB * GPB, IW), jnp.int32),
            pltpu.VMEM((CB * L, D), jnp.float32),
            pltpu.VMEM((CB, D), jnp.float32),
            pltpu.SemaphoreType.DMA,
        ],
        compiler_params=pltpu.CompilerParams(use_tc_tiling_on_sc=False),
    )
    return k(x2d, emb_table)


def _tc_linear(means, Wt, b2):
    def body(s_ref, w_ref, b_ref, o_ref):
        o_ref[...] = (
            jnp.dot(s_ref[...], w_ref[...], preferred_element_type=jnp.float32)
            + b_ref[...]
        )

    BLK = 4096
    return pl.pallas_call(
        body,
        grid=(B // BLK,),
        in_specs=[
            pl.BlockSpec((BLK, D), lambda i: (i, 0)),
            pl.BlockSpec((D, D), lambda i: (0, 0)),
            pl.BlockSpec((1, D), lambda i: (0, 0)),
        ],
        out_specs=pl.BlockSpec((BLK, D), lambda i: (i, 0)),
        out_shape=jax.ShapeDtypeStruct((B, D), jnp.float32),
    )(means, Wt, b2)


def kernel(x, emb_table, W, b):
    x2d = x.astype(jnp.int32).reshape(B * GPB, IW)
    means = _bag_means(x2d, emb_table)
    return _tc_linear(means, W.T, b.reshape(1, D))


# speedup vs baseline: 3.1573x; 1.3349x over previous
import functools

import jax
import jax.numpy as jnp
from jax import lax
from jax.experimental import pallas as pl
from jax.experimental.pallas import tpu as pltpu
from jax.experimental.pallas import tpu_sc as plsc

VOCAB = 1_000_000
D = 64
B = 16384
L = 200
NW = 32
BPW = B // NW
CB = 4
NCHUNK = BPW // CB
IW = 100
GPB = L // IW
CPS = 16
NSUPER = NCHUNK // CPS
CROWS = CB * L
UNROLL = 4


def _sc_body(x_hbm, tab_hbm, out_hbm, idx_v, rows0, rows1, out_v, sem0, sem1):
    cid = lax.axis_index("c")
    sid = lax.axis_index("s")
    wid = sid * 2 + cid
    wbase = wid * BPW
    inv = jnp.float32(1.0 / L)
    bufs = ((rows0, sem0), (rows1, sem1))

    def fire(c_local, rows, sem):
        for j in range(CB * GPB):
            pltpu.async_copy(
                tab_hbm.at[idx_v.at[c_local * CB * GPB + j]],
                rows.at[pl.ds(j * IW, IW)],
                sem,
            )

    def drain(rows, sem):
        pltpu.make_async_copy(tab_hbm.at[pl.ds(0, CROWS)], rows, sem).wait()

    def reduce(rows, bag0):
        for i in range(CB):
            base = i * L

            def body(r, accs):
                a0, a1, a2, a3 = accs
                row = base + r * UNROLL
                for u in range(UNROLL):
                    a0 = a0 + rows[row + u, pl.ds(0, 16)]
                    a1 = a1 + rows[row + u, pl.ds(16, 16)]
                    a2 = a2 + rows[row + u, pl.ds(32, 16)]
                    a3 = a3 + rows[row + u, pl.ds(48, 16)]
                return (a0, a1, a2, a3)

            z = jnp.zeros((16,), jnp.float32)
            a0, a1, a2, a3 = lax.fori_loop(0, L // UNROLL, body, (z, z, z, z))
            out_v[i, pl.ds(0, 16)] = a0 * inv
            out_v[i, pl.ds(16, 16)] = a1 * inv
            out_v[i, pl.ds(32, 16)] = a2 * inv
            out_v[i, pl.ds(48, 16)] = a3 * inv
        pltpu.sync_copy(out_v, out_hbm.at[pl.ds(bag0, CB)])

    def super_body(g, carry):
        sbag0 = wbase + g * CPS * CB
        pltpu.sync_copy(x_hbm.at[pl.ds(sbag0 * GPB, CPS * CB * GPB)], idx_v)
        fire(0, *bufs[0])
        for c in range(CPS):
            p = c % 2
            if c + 1 < CPS:
                fire(c + 1, *bufs[1 - p])
            drain(*bufs[p])
            reduce(bufs[p][0], sbag0 + c * CB)
        return carry

    lax.fori_loop(0, NSUPER, super_body, 0)


def _bag_means(x2d, emb_table):
    mesh = plsc.VectorSubcoreMesh(core_axis_name="c", subcore_axis_name="s")
    k = pl.kernel(
        _sc_body,
        out_type=jax.ShapeDtypeStruct((B, D), jnp.float32),
        mesh=mesh,
        scratch_types=[
            pltpu.VMEM((CPS * CB * GPB, IW), jnp.int32),
            pltpu.VMEM((CROWS, D), jnp.float32),
            pltpu.VMEM((CROWS, D), jnp.float32),
            pltpu.VMEM((CB, D), jnp.float32),
            pltpu.SemaphoreType.DMA,
            pltpu.SemaphoreType.DMA,
        ],
        compiler_params=pltpu.CompilerParams(use_tc_tiling_on_sc=False),
    )
    return k(x2d, emb_table)


def _tc_linear(means, Wt, b2):
    def body(s_ref, w_ref, b_ref, o_ref):
        o_ref[...] = (
            jnp.dot(s_ref[...], w_ref[...], preferred_element_type=jnp.float32)
            + b_ref[...]
        )

    BLK = 4096
    return pl.pallas_call(
        body,
        grid=(B // BLK,),
        in_specs=[
            pl.BlockSpec((BLK, D), lambda i: (i, 0)),
            pl.BlockSpec((D, D), lambda i: (0, 0)),
            pl.BlockSpec((1, D), lambda i: (0, 0)),
        ],
        out_specs=pl.BlockSpec((BLK, D), lambda i: (i, 0)),
        out_shape=jax.ShapeDtypeStruct((B, D), jnp.float32),
    )(means, Wt, b2)


def kernel(x, emb_table, W, b):
    x2d = x.astype(jnp.int32).reshape(B * GPB, IW)
    means = _bag_means(x2d, emb_table)
    return _tc_linear(means, W.T, b.reshape(1, D))


# speedup vs baseline: 3.3231x; 1.0525x over previous
import functools

import jax
import jax.numpy as jnp
from jax import lax
from jax.experimental import pallas as pl
from jax.experimental.pallas import tpu as pltpu
from jax.experimental.pallas import tpu_sc as plsc

VOCAB = 1_000_000
D = 64
B = 16384
L = 200
NW = 32
BPW = B // NW
CB = 2
NCHUNK = BPW // CB
SPLITS = ((0, 128), (128, 72))
GPB = len(SPLITS)
CPS = 32
SB = CPS * CB
NSUPER = NCHUNK // CPS
CROWS = CB * L
NBUF = 4
UNROLL = 4


def _sc_body(x_hbm, tab_hbm, out_hbm, idx_v, r0, r1, r2, r3, out_v,
             s0, s1, s2, s3):
    cid = lax.axis_index("c")
    sid = lax.axis_index("s")
    wid = sid * 2 + cid
    wbase = wid * BPW
    inv = jnp.float32(1.0 / L)
    bufs = ((r0, s0), (r1, s1), (r2, s2), (r3, s3))

    def fire(c_local, rows, sem):
        for i in range(CB):
            for off, n in SPLITS:
                pltpu.async_copy(
                    tab_hbm.at[idx_v.at[c_local * CB + i, pl.ds(off, n)]],
                    rows.at[pl.ds(i * L + off, n)],
                    sem,
                )

    def drain(rows, sem):
        pltpu.make_async_copy(tab_hbm.at[pl.ds(0, CROWS)], rows, sem).wait()

    def reduce(rows, obag0):
        for i in range(CB):
            base = i * L

            def body(r, accs):
                a0, a1, a2, a3 = accs
                row = base + r * UNROLL
                for u in range(UNROLL):
                    a0 = a0 + rows[row + u, pl.ds(0, 16)]
                    a1 = a1 + rows[row + u, pl.ds(16, 16)]
                    a2 = a2 + rows[row + u, pl.ds(32, 16)]
                    a3 = a3 + rows[row + u, pl.ds(48, 16)]
                return (a0, a1, a2, a3)

            z = jnp.zeros((16,), jnp.float32)
            a0, a1, a2, a3 = lax.fori_loop(0, L // UNROLL, body, (z, z, z, z))
            out_v[obag0 + i, pl.ds(0, 16)] = a0 * inv
            out_v[obag0 + i, pl.ds(16, 16)] = a1 * inv
            out_v[obag0 + i, pl.ds(32, 16)] = a2 * inv
            out_v[obag0 + i, pl.ds(48, 16)] = a3 * inv

    def super_body(g, carry):
        sbag0 = wbase + g * SB
        pltpu.sync_copy(x_hbm.at[pl.ds(sbag0, SB)], idx_v)
        for c in range(NBUF - 1):
            fire(c, *bufs[c])
        for c in range(CPS):
            p = c % NBUF
            if c + NBUF - 1 < CPS:
                fire(c + NBUF - 1, *bufs[(c + NBUF - 1) % NBUF])
            drain(*bufs[p])
            reduce(bufs[p][0], c * CB)
        pltpu.sync_copy(out_v, out_hbm.at[pl.ds(sbag0, SB)])
        return carry

    lax.fori_loop(0, NSUPER, super_body, 0)


def _bag_means(x, emb_table):
    mesh = plsc.VectorSubcoreMesh(core_axis_name="c", subcore_axis_name="s")
    k = pl.kernel(
        _sc_body,
        out_type=jax.ShapeDtypeStruct((B, D), jnp.float32),
        mesh=mesh,
        scratch_types=[
            pltpu.VMEM((SB, L), jnp.int32),
            pltpu.VMEM((CROWS, D), jnp.float32),
            pltpu.VMEM((CROWS, D), jnp.float32),
            pltpu.VMEM((CROWS, D), jnp.float32),
            pltpu.VMEM((CROWS, D), jnp.float32),
            pltpu.VMEM((SB, D), jnp.float32),
            pltpu.SemaphoreType.DMA,
            pltpu.SemaphoreType.DMA,
            pltpu.SemaphoreType.DMA,
            pltpu.SemaphoreType.DMA,
        ],
        compiler_params=pltpu.CompilerParams(use_tc_tiling_on_sc=False),
    )
    return k(x, emb_table)


def _tc_linear(means, Wt, b2):
    def body(s_ref, w_ref, b_ref, o_ref):
        o_ref[...] = (
            jnp.dot(s_ref[...], w_ref[...], preferred_element_type=jnp.float32)
            + b_ref[...]
        )

    BLK = 4096
    return pl.pallas_call(
        body,
        grid=(B // BLK,),
        in_specs=[
            pl.BlockSpec((BLK, D), lambda i: (i, 0)),
            pl.BlockSpec((D, D), lambda i: (0, 0)),
            pl.BlockSpec((1, D), lambda i: (0, 0)),
        ],
        out_specs=pl.BlockSpec((BLK, D), lambda i: (i, 0)),
        out_shape=jax.ShapeDtypeStruct((B, D), jnp.float32),
    )(means, Wt, b2)


def kernel(x, emb_table, W, b):
    means = _bag_means(x.astype(jnp.int32), emb_table)
    return _tc_linear(means, W.T, b.reshape(1, D))
